# baseline (speedup 1.0000x reference)
import functools

import jax
import jax.numpy as jnp
from jax import lax
from jax.experimental import pallas as pl
from jax.experimental.pallas import tpu as pltpu
from jax.experimental.pallas import tpu_sc as plsc

N = 10000
E = 320000
D = 128
L = 4
C = 10

CH = 64
NB = 5
GA = NB - 1


def _make_agg(nc, ns, nch0, nch1):
    nchmax = max(nch0, nch1)
    rows_per_tile = -(-(N + 1) // (ns * 16)) * 16
    acc_rows = rows_per_tile * ns
    out_rows_pt = (N // ns) // 8 * 8
    out_rows_last = N - out_rows_pt * (ns - 1)

    mesh = plsc.VectorSubcoreMesh(core_axis_name="c", subcore_axis_name="s")

    @functools.partial(
        pl.kernel,
        mesh=mesh,
        out_type=jax.ShapeDtypeStruct((nc, N, D), jnp.float32),
        scratch_types=[
            pltpu.VMEM((NB, CH), jnp.int32),
            pltpu.VMEM((NB, CH), jnp.int32),
            pltpu.VMEM((NB, CH, D), jnp.float32),
            pltpu.VMEM((16, D), jnp.float32),
            pltpu.VMEM_SHARED((acc_rows, D), jnp.float32),
            pltpu.SemaphoreType.DMA,
            pltpu.SemaphoreType.DMA,
            pltpu.SemaphoreType.DMA,
            pltpu.SemaphoreType.DMA,
            pltpu.SemaphoreType.DMA,
        ],
    )
    def agg(src_hbm, dst_hbm, h_hbm, out_hbm, src_v, dst_v, rows_v, zero_v,
            acc_sh, gsem, isem, dsem, ssem, zsem):
        c = lax.axis_index("c")
        s = lax.axis_index("s")
        nchunk = jnp.where(c == 0, nch0, nch1)

        for k in range(GA):
            pltpu.sync_copy(src_hbm.at[c, s, k], src_v.at[k])
        pltpu.sync_copy(dst_hbm.at[c, s, 0], dst_v.at[0])
        pltpu.sync_copy(dst_hbm.at[c, s, 1], dst_v.at[1])
        for k in range(GA):
            pltpu.async_copy(h_hbm.at[src_v.at[k]], rows_v.at[k], gsem)
        for k in range(GA, NB):
            pltpu.async_copy(src_hbm.at[c, s, k], src_v.at[k], isem)
        for k in range(2, GA):
            pltpu.async_copy(dst_hbm.at[c, s, k], dst_v.at[k], dsem)

        for r in range(16):
            for q in range(D // 16):
                zero_v[r, pl.ds(q * 16, 16)] = jnp.zeros((16,), jnp.float32)

        def zbody(i, carry):
            zoff = pl.multiple_of(s * rows_per_tile + i * 16, 16)
            pltpu.async_copy(zero_v, acc_sh.at[pl.ds(zoff, 16)], zsem)
            return carry

        lax.fori_loop(0, rows_per_tile // 16, zbody, 0)

        def zdrain(i, carry):
            zoff = pl.multiple_of(s * rows_per_tile + i * 16, 16)
            pltpu.make_async_copy(zero_v, acc_sh.at[pl.ds(zoff, 16)],
                                  zsem).wait()
            return carry

        lax.fori_loop(0, rows_per_tile // 16, zdrain, 0)
        plsc.subcore_barrier()

        def body(j, carry):
            b = lax.rem(j, NB)
            pltpu.make_async_copy(h_hbm.at[src_v.at[b]], rows_v.at[b],
                                  gsem).wait()

            @pl.when(j >= 1)
            def _():
                pb = lax.rem(j + NB - 1, NB)
                pltpu.make_async_copy(rows_v.at[pb], acc_sh.at[dst_v.at[pb]],
                                      ssem).wait()

            @pl.when(j + GA < nchunk)
            def _():
                b2 = lax.rem(j + GA, NB)
                pltpu.make_async_copy(src_hbm.at[c, s, j + GA], src_v.at[b2],
                                      isem).wait()
                pltpu.async_copy(h_hbm.at[src_v.at[b2]], rows_v.at[b2], gsem)

            @pl.when(j >= 2)
            def _():
                pltpu.make_async_copy(dst_hbm.at[c, s, j], dst_v.at[b],
                                      dsem).wait()

            pltpu.async_copy(rows_v.at[b], acc_sh.at[dst_v.at[b]], ssem,
                             add=True)

            @pl.when(j + NB < nchunk)
            def _():
                pltpu.async_copy(src_hbm.at[c, s, j + NB], src_v.at[b], isem)

            @pl.when(j + GA < nchunk)
            def _():
                b3 = lax.rem(j + GA, NB)
                pltpu.async_copy(dst_hbm.at[c, s, j + GA], dst_v.at[b3], dsem)

            return carry

        lax.fori_loop(0, nchunk, body, 0)
        lb = lax.rem(nchunk - 1, NB)
        pltpu.make_async_copy(rows_v.at[lb], acc_sh.at[dst_v.at[lb]],
                              ssem).wait()
        plsc.subcore_barrier()
        ooff = pl.multiple_of(s * out_rows_pt, 8)

        @pl.when(s < ns - 1)
        def _():
            pltpu.sync_copy(acc_sh.at[pl.ds(ooff, out_rows_pt)],
                            out_hbm.at[c, pl.ds(ooff, out_rows_pt)])

        @pl.when(s == ns - 1)
        def _():
            loff = pl.multiple_of((ns - 1) * out_rows_pt, 8)
            pltpu.sync_copy(acc_sh.at[pl.ds(loff, out_rows_last)],
                            out_hbm.at[c, pl.ds(loff, out_rows_last)])

    return agg


def _embed_body(h_ref, w_ref, o_ref):
    o_ref[...] = jnp.dot(h_ref[...], w_ref[...],
                         preferred_element_type=jnp.float32)


def _bn(x, gamma, beta):
    mu = jnp.mean(x, axis=0, keepdims=True)
    var = jnp.mean((x - mu) ** 2, axis=0, keepdims=True)
    return (x - mu) * lax.rsqrt(var + 1e-5) * gamma + beta


def _layer_core(epsp1_ref, h_ref, agg_ref, sn_ref, w1_ref, b1_ref, g1_ref,
                be1_ref, w2_ref, b2_ref, ga_ref, ba_ref, gn_ref, bn_ref):
    x = epsp1_ref[...] * h_ref[...] + agg_ref[0] + agg_ref[1]
    t = jnp.dot(x, w1_ref[...], preferred_element_type=jnp.float32) + b1_ref[...]
    t = jnp.maximum(_bn(t, g1_ref[...], be1_ref[...]), 0.0)
    u = jnp.dot(t, w2_ref[...], preferred_element_type=jnp.float32) + b2_ref[...]
    u = jnp.maximum(_bn(u, ga_ref[...], ba_ref[...]), 0.0)
    u = u * sn_ref[...]
    u = jnp.maximum(_bn(u, gn_ref[...], bn_ref[...]), 0.0)
    return u


def _mid_body(epsp1_ref, h_ref, agg_ref, sn_ref, w1_ref, b1_ref, g1_ref,
              be1_ref, w2_ref, b2_ref, ga_ref, ba_ref, gn_ref, bn_ref,
              hin_ref, o_ref):
    u = _layer_core(epsp1_ref, h_ref, agg_ref, sn_ref, w1_ref, b1_ref, g1_ref,
                    be1_ref, w2_ref, b2_ref, ga_ref, ba_ref, gn_ref, bn_ref)
    o_ref[...] = u + hin_ref[...]


def _last_body(epsp1_ref, h_ref, agg_ref, sn_ref, w1_ref, b1_ref, g1_ref,
               be1_ref, w2_ref, b2_ref, ga_ref, ba_ref, gn_ref, bn_ref,
               hin_ref, wro_ref, wpred_ref, bp_ref, o_ref):
    u = _layer_core(epsp1_ref, h_ref, agg_ref, sn_ref, w1_ref, b1_ref, g1_ref,
                    be1_ref, w2_ref, b2_ref, ga_ref, ba_ref, gn_ref, bn_ref)
    hf = u + hin_ref[...]
    hm = jnp.mean(hf, axis=0, keepdims=True)
    t1 = jnp.dot(hm, wro_ref[...], preferred_element_type=jnp.float32)
    o_ref[...] = jnp.dot(t1, wpred_ref[...],
                         preferred_element_type=jnp.float32) + bp_ref[...]


def kernel(g, h, e, snorm_n, snorm_e, W_embed, eps, W1, b1, gamma1, beta1,
           W2, b2, gamma_a, beta_a, gamma_n, beta_n, W_ro, W_pred, b_pred):
    info = plsc.get_sparse_core_info()
    nc, ns = info.num_cores, info.num_subcores
    total_chunks = -(-E // (ns * CH))
    nch0 = max(NB, round(total_chunks * 0.655))
    nch1 = total_chunks + 1 - nch0
    nchmax = max(nch0, nch1)
    agg_fn = _make_agg(nc, ns, nch0, nch1)

    src = g[0].astype(jnp.int32)
    dst = g[1].astype(jnp.int32)
    e0 = ns * CH * nch0
    e1 = ns * CH * nch1
    pad = e0 + e1 - E
    src_p = jnp.concatenate([src, jnp.zeros((pad,), jnp.int32)])
    dst_p = jnp.concatenate([dst, jnp.full((pad,), N, jnp.int32)])

    def to4(x):
        a = x[:e0].reshape(ns, nch0, CH)
        b = x[e0:].reshape(ns, nch1, CH)
        a = jnp.pad(a, ((0, 0), (0, nchmax - nch0), (0, 0)))
        b = jnp.pad(b, ((0, 0), (0, nchmax - nch1), (0, 0)))
        return jnp.stack([a, b])

    src3 = to4(src_p)
    dst3 = to4(dst_p)

    hcur = pl.pallas_call(
        _embed_body,
        out_shape=jax.ShapeDtypeStruct((N, D), jnp.float32),
    )(h.astype(jnp.float32), W_embed)
    h0 = hcur

    score = None
    for i in range(L):
        agg2 = agg_fn(src3, dst3, hcur)
        epsp1 = (1.0 + eps[i]).reshape(1, 1)
        args = (epsp1, hcur, agg2, snorm_n, W1[i], b1[i].reshape(1, D),
                gamma1[i].reshape(1, D), beta1[i].reshape(1, D), W2[i],
                b2[i].reshape(1, D), gamma_a[i].reshape(1, D),
                beta_a[i].reshape(1, D), gamma_n[i].reshape(1, D),
                beta_n[i].reshape(1, D))
        if i < L - 1:
            hcur = pl.pallas_call(
                _mid_body,
                out_shape=jax.ShapeDtypeStruct((N, D), jnp.float32),
            )(*args, h0)
        else:
            score = pl.pallas_call(
                _last_body,
                out_shape=jax.ShapeDtypeStruct((1, C), jnp.float32),
            )(*args, h0, W_ro, W_pred, b_pred.reshape(1, C))
    return score


# speedup vs baseline: 1.0213x; 1.0213x over previous
import functools

import jax
import jax.numpy as jnp
from jax import lax
from jax.experimental import pallas as pl
from jax.experimental.pallas import tpu as pltpu
from jax.experimental.pallas import tpu_sc as plsc

N = 10000
E = 320000
D = 128
L = 4
C = 10

CH = 64
NB = 5
GA = NB - 2


def _make_agg(nc, ns, nch0, nch1):
    nchmax = max(nch0, nch1)
    rows_per_tile = -(-(N + 1) // (ns * 16)) * 16
    acc_rows = rows_per_tile * ns
    out_rows_pt = (N // ns) // 8 * 8
    out_rows_last = N - out_rows_pt * (ns - 1)

    mesh = plsc.VectorSubcoreMesh(core_axis_name="c", subcore_axis_name="s")

    @functools.partial(
        pl.kernel,
        mesh=mesh,
        out_type=jax.ShapeDtypeStruct((nc, N, D), jnp.float32),
        scratch_types=[
            pltpu.VMEM((NB, CH), jnp.int32),
            pltpu.VMEM((NB, CH), jnp.int32),
            pltpu.VMEM((NB, CH, D), jnp.float32),
            pltpu.VMEM((16, D), jnp.float32),
            pltpu.VMEM_SHARED((acc_rows, D), jnp.float32),
            pltpu.SemaphoreType.DMA,
            pltpu.SemaphoreType.DMA,
            pltpu.SemaphoreType.DMA,
            pltpu.SemaphoreType.DMA,
            pltpu.SemaphoreType.DMA,
        ],
    )
    def agg(src_hbm, dst_hbm, h_hbm, out_hbm, src_v, dst_v, rows_v, zero_v,
            acc_sh, gsem, isem, dsem, ssem, zsem):
        c = lax.axis_index("c")
        s = lax.axis_index("s")
        nchunk = jnp.where(c == 0, nch0, nch1)

        for k in range(GA):
            pltpu.sync_copy(src_hbm.at[c, s, k], src_v.at[k])
        pltpu.sync_copy(dst_hbm.at[c, s, 0], dst_v.at[0])
        pltpu.sync_copy(dst_hbm.at[c, s, 1], dst_v.at[1])
        for k in range(GA):
            pltpu.async_copy(h_hbm.at[src_v.at[k]], rows_v.at[k], gsem)
        for k in range(GA, NB):
            pltpu.async_copy(src_hbm.at[c, s, k], src_v.at[k], isem)
        for k in range(2, GA):
            pltpu.async_copy(dst_hbm.at[c, s, k], dst_v.at[k], dsem)

        for r in range(16):
            for q in range(D // 16):
                zero_v[r, pl.ds(q * 16, 16)] = jnp.zeros((16,), jnp.float32)

        def zbody(i, carry):
            zoff = pl.multiple_of(s * rows_per_tile + i * 16, 16)
            pltpu.async_copy(zero_v, acc_sh.at[pl.ds(zoff, 16)], zsem)
            return carry

        lax.fori_loop(0, rows_per_tile // 16, zbody, 0)

        def zdrain(i, carry):
            zoff = pl.multiple_of(s * rows_per_tile + i * 16, 16)
            pltpu.make_async_copy(zero_v, acc_sh.at[pl.ds(zoff, 16)],
                                  zsem).wait()
            return carry

        lax.fori_loop(0, rows_per_tile // 16, zdrain, 0)
        plsc.subcore_barrier()

        def body(j, carry):
            b = lax.rem(j, NB)
            pltpu.make_async_copy(h_hbm.at[src_v.at[b]], rows_v.at[b],
                                  gsem).wait()

            @pl.when(j >= 1)
            def _():
                pb = lax.rem(j + NB - 1, NB)
                pltpu.make_async_copy(rows_v.at[pb], acc_sh.at[dst_v.at[pb]],
                                      ssem).wait()

            @pl.when(j + GA < nchunk)
            def _():
                b2 = lax.rem(j + GA, NB)
                pltpu.make_async_copy(src_hbm.at[c, s, j + GA], src_v.at[b2],
                                      isem).wait()
                pltpu.async_copy(h_hbm.at[src_v.at[b2]], rows_v.at[b2], gsem)

            @pl.when(j >= 2)
            def _():
                pltpu.make_async_copy(dst_hbm.at[c, s, j], dst_v.at[b],
                                      dsem).wait()

            pltpu.async_copy(rows_v.at[b], acc_sh.at[dst_v.at[b]], ssem,
                             add=True)

            @pl.when(j + NB < nchunk)
            def _():
                pltpu.async_copy(src_hbm.at[c, s, j + NB], src_v.at[b], isem)

            @pl.when(j + GA < nchunk)
            def _():
                b3 = lax.rem(j + GA, NB)
                pltpu.async_copy(dst_hbm.at[c, s, j + GA], dst_v.at[b3], dsem)

            return carry

        lax.fori_loop(0, nchunk, body, 0)
        lb = lax.rem(nchunk - 1, NB)
        pltpu.make_async_copy(rows_v.at[lb], acc_sh.at[dst_v.at[lb]],
                              ssem).wait()
        plsc.subcore_barrier()
        ooff = pl.multiple_of(s * out_rows_pt, 8)

        @pl.when(s < ns - 1)
        def _():
            pltpu.sync_copy(acc_sh.at[pl.ds(ooff, out_rows_pt)],
                            out_hbm.at[c, pl.ds(ooff, out_rows_pt)])

        @pl.when(s == ns - 1)
        def _():
            loff = pl.multiple_of((ns - 1) * out_rows_pt, 8)
            pltpu.sync_copy(acc_sh.at[pl.ds(loff, out_rows_last)],
                            out_hbm.at[c, pl.ds(loff, out_rows_last)])

    return agg


def _embed_body(h_ref, w_ref, o_ref):
    o_ref[...] = jnp.dot(h_ref[...], w_ref[...],
                         preferred_element_type=jnp.float32)


def _bn(x, gamma, beta):
    mu = jnp.mean(x, axis=0, keepdims=True)
    var = jnp.mean((x - mu) ** 2, axis=0, keepdims=True)
    return (x - mu) * lax.rsqrt(var + 1e-5) * gamma + beta


def _layer_core(epsp1_ref, h_ref, agg_ref, sn_ref, w1_ref, b1_ref, g1_ref,
                be1_ref, w2_ref, b2_ref, ga_ref, ba_ref, gn_ref, bn_ref):
    x = epsp1_ref[...] * h_ref[...] + agg_ref[0] + agg_ref[1]
    t = jnp.dot(x, w1_ref[...], preferred_element_type=jnp.float32) + b1_ref[...]
    t = jnp.maximum(_bn(t, g1_ref[...], be1_ref[...]), 0.0)
    u = jnp.dot(t, w2_ref[...], preferred_element_type=jnp.float32) + b2_ref[...]
    u = jnp.maximum(_bn(u, ga_ref[...], ba_ref[...]), 0.0)
    u = u * sn_ref[...]
    u = jnp.maximum(_bn(u, gn_ref[...], bn_ref[...]), 0.0)
    return u


def _mid_body(epsp1_ref, h_ref, agg_ref, sn_ref, w1_ref, b1_ref, g1_ref,
              be1_ref, w2_ref, b2_ref, ga_ref, ba_ref, gn_ref, bn_ref,
              hin_ref, o_ref):
    u = _layer_core(epsp1_ref, h_ref, agg_ref, sn_ref, w1_ref, b1_ref, g1_ref,
                    be1_ref, w2_ref, b2_ref, ga_ref, ba_ref, gn_ref, bn_ref)
    o_ref[...] = u + hin_ref[...]


def _last_body(epsp1_ref, h_ref, agg_ref, sn_ref, w1_ref, b1_ref, g1_ref,
               be1_ref, w2_ref, b2_ref, ga_ref, ba_ref, gn_ref, bn_ref,
               hin_ref, wro_ref, wpred_ref, bp_ref, o_ref):
    u = _layer_core(epsp1_ref, h_ref, agg_ref, sn_ref, w1_ref, b1_ref, g1_ref,
                    be1_ref, w2_ref, b2_ref, ga_ref, ba_ref, gn_ref, bn_ref)
    hf = u + hin_ref[...]
    hm = jnp.mean(hf, axis=0, keepdims=True)
    t1 = jnp.dot(hm, wro_ref[...], preferred_element_type=jnp.float32)
    o_ref[...] = jnp.dot(t1, wpred_ref[...],
                         preferred_element_type=jnp.float32) + bp_ref[...]


def kernel(g, h, e, snorm_n, snorm_e, W_embed, eps, W1, b1, gamma1, beta1,
           W2, b2, gamma_a, beta_a, gamma_n, beta_n, W_ro, W_pred, b_pred):
    info = plsc.get_sparse_core_info()
    nc, ns = info.num_cores, info.num_subcores
    total_chunks = -(-E // (ns * CH))
    nch0 = max(NB, round(total_chunks * 0.655))
    nch1 = total_chunks + 1 - nch0
    nchmax = max(nch0, nch1)
    agg_fn = _make_agg(nc, ns, nch0, nch1)

    src = g[0].astype(jnp.int32)
    dst = g[1].astype(jnp.int32)
    e0 = ns * CH * nch0
    e1 = ns * CH * nch1
    pad = e0 + e1 - E
    src_p = jnp.concatenate([src, jnp.zeros((pad,), jnp.int32)])
    dst_p = jnp.concatenate([dst, jnp.full((pad,), N, jnp.int32)])

    def to4(x):
        a = x[:e0].reshape(ns, nch0, CH)
        b = x[e0:].reshape(ns, nch1, CH)
        a = jnp.pad(a, ((0, 0), (0, nchmax - nch0), (0, 0)))
        b = jnp.pad(b, ((0, 0), (0, nchmax - nch1), (0, 0)))
        return jnp.stack([a, b])

    src3 = to4(src_p)
    dst3 = to4(dst_p)

    hcur = pl.pallas_call(
        _embed_body,
        out_shape=jax.ShapeDtypeStruct((N, D), jnp.float32),
    )(h.astype(jnp.float32), W_embed)
    h0 = hcur

    score = None
    for i in range(L):
        agg2 = agg_fn(src3, dst3, hcur)
        epsp1 = (1.0 + eps[i]).reshape(1, 1)
        args = (epsp1, hcur, agg2, snorm_n, W1[i], b1[i].reshape(1, D),
                gamma1[i].reshape(1, D), beta1[i].reshape(1, D), W2[i],
                b2[i].reshape(1, D), gamma_a[i].reshape(1, D),
                beta_a[i].reshape(1, D), gamma_n[i].reshape(1, D),
                beta_n[i].reshape(1, D))
        if i < L - 1:
            hcur = pl.pallas_call(
                _mid_body,
                out_shape=jax.ShapeDtypeStruct((N, D), jnp.float32),
            )(*args, h0)
        else:
            score = pl.pallas_call(
                _last_body,
                out_shape=jax.ShapeDtypeStruct((1, C), jnp.float32),
            )(*args, h0, W_ro, W_pred, b_pred.reshape(1, C))
    return score


# speedup vs baseline: 1.0373x; 1.0157x over previous
import functools

import jax
import jax.numpy as jnp
from jax import lax
from jax.experimental import pallas as pl
from jax.experimental.pallas import tpu as pltpu
from jax.experimental.pallas import tpu_sc as plsc

N = 10000
E = 320000
D = 128
L = 4
C = 10

CH = 64
NB = 5
GA = NB - 2


def _make_agg(nc, ns, nch0, nch1):
    nchmax = max(nch0, nch1)
    rows_per_tile = -(-(N + 1) // (ns * 16)) * 16
    acc_rows = rows_per_tile * ns
    out_rows_pt = (N // ns) // 8 * 8
    out_rows_last = N - out_rows_pt * (ns - 1)

    mesh = plsc.VectorSubcoreMesh(core_axis_name="c", subcore_axis_name="s")

    @functools.partial(
        pl.kernel,
        mesh=mesh,
        out_type=jax.ShapeDtypeStruct((nc, N, D), jnp.float32),
        scratch_types=[
            pltpu.VMEM((NB, CH), jnp.int32),
            pltpu.VMEM((NB, CH), jnp.int32),
            pltpu.VMEM((NB, CH, D), jnp.float32),
            pltpu.VMEM((16, D), jnp.float32),
            pltpu.VMEM_SHARED((acc_rows, D), jnp.float32),
            pltpu.SemaphoreType.DMA,
            pltpu.SemaphoreType.DMA,
            pltpu.SemaphoreType.DMA,
            pltpu.SemaphoreType.DMA,
            pltpu.SemaphoreType.DMA,
        ],
    )
    def agg(src_hbm, dst_hbm, h_hbm, out_hbm, src_v, dst_v, rows_v, zero_v,
            acc_sh, gsem, isem, dsem, ssem, zsem):
        c = lax.axis_index("c")
        s = lax.axis_index("s")
        nchunk = jnp.where(c == 0, nch0, nch1)

        for k in range(GA):
            pltpu.sync_copy(src_hbm.at[c, s, k], src_v.at[k])
        pltpu.sync_copy(dst_hbm.at[c, s, 0], dst_v.at[0])
        pltpu.sync_copy(dst_hbm.at[c, s, 1], dst_v.at[1])
        for k in range(GA):
            pltpu.async_copy(h_hbm.at[src_v.at[k]], rows_v.at[k], gsem)
        for k in range(GA, NB):
            pltpu.async_copy(src_hbm.at[c, s, k], src_v.at[k], isem)
        for k in range(2, GA):
            pltpu.async_copy(dst_hbm.at[c, s, k], dst_v.at[k], dsem)

        for r in range(16):
            for q in range(D // 16):
                zero_v[r, pl.ds(q * 16, 16)] = jnp.zeros((16,), jnp.float32)

        def zbody(i, carry):
            zoff = pl.multiple_of(s * rows_per_tile + i * 16, 16)
            pltpu.async_copy(zero_v, acc_sh.at[pl.ds(zoff, 16)], zsem)
            return carry

        lax.fori_loop(0, rows_per_tile // 16, zbody, 0)

        def zdrain(i, carry):
            zoff = pl.multiple_of(s * rows_per_tile + i * 16, 16)
            pltpu.make_async_copy(zero_v, acc_sh.at[pl.ds(zoff, 16)],
                                  zsem).wait()
            return carry

        lax.fori_loop(0, rows_per_tile // 16, zdrain, 0)
        plsc.subcore_barrier()

        def body(j, carry):
            b = lax.rem(j, NB)
            pltpu.make_async_copy(h_hbm.at[src_v.at[b]], rows_v.at[b],
                                  gsem).wait()

            @pl.when(j >= 1)
            def _():
                pb = lax.rem(j + NB - 1, NB)
                pltpu.make_async_copy(rows_v.at[pb], acc_sh.at[dst_v.at[pb]],
                                      ssem).wait()

            @pl.when(j + GA < nchunk)
            def _():
                b2 = lax.rem(j + GA, NB)
                pltpu.make_async_copy(src_hbm.at[c, s, j + GA], src_v.at[b2],
                                      isem).wait()
                pltpu.async_copy(h_hbm.at[src_v.at[b2]], rows_v.at[b2], gsem)

            @pl.when(j >= 2)
            def _():
                pltpu.make_async_copy(dst_hbm.at[c, s, j], dst_v.at[b],
                                      dsem).wait()

            pltpu.async_copy(rows_v.at[b], acc_sh.at[dst_v.at[b]], ssem,
                             add=True)

            @pl.when(j + NB < nchunk)
            def _():
                pltpu.async_copy(src_hbm.at[c, s, j + NB], src_v.at[b], isem)

            @pl.when(j + GA < nchunk)
            def _():
                b3 = lax.rem(j + GA, NB)
                pltpu.async_copy(dst_hbm.at[c, s, j + GA], dst_v.at[b3], dsem)

            return carry

        lax.fori_loop(0, nchunk, body, 0)
        lb = lax.rem(nchunk - 1, NB)
        pltpu.make_async_copy(rows_v.at[lb], acc_sh.at[dst_v.at[lb]],
                              ssem).wait()
        plsc.subcore_barrier()
        ooff = pl.multiple_of(s * out_rows_pt, 8)

        @pl.when(s < ns - 1)
        def _():
            pltpu.sync_copy(acc_sh.at[pl.ds(ooff, out_rows_pt)],
                            out_hbm.at[c, pl.ds(ooff, out_rows_pt)])

        @pl.when(s == ns - 1)
        def _():
            loff = pl.multiple_of((ns - 1) * out_rows_pt, 8)
            pltpu.sync_copy(acc_sh.at[pl.ds(loff, out_rows_last)],
                            out_hbm.at[c, pl.ds(loff, out_rows_last)])

    return agg


def _embed_body(h_ref, w_ref, o_ref):
    o_ref[...] = jnp.dot(h_ref[...], w_ref[...],
                         preferred_element_type=jnp.float32)


def _bn(x, gamma, beta):
    mu = jnp.mean(x, axis=0, keepdims=True)
    m2 = jnp.mean(x * x, axis=0, keepdims=True)
    var = m2 - mu * mu
    return (x - mu) * lax.rsqrt(var + 1e-5) * gamma + beta


def _layer_core(epsp1_ref, h_ref, agg_ref, sn_ref, w1_ref, b1_ref, g1_ref,
                be1_ref, w2_ref, b2_ref, ga_ref, ba_ref, gn_ref, bn_ref):
    x = epsp1_ref[...] * h_ref[...] + agg_ref[0] + agg_ref[1]
    t = jnp.dot(x, w1_ref[...], preferred_element_type=jnp.float32) + b1_ref[...]
    t = jnp.maximum(_bn(t, g1_ref[...], be1_ref[...]), 0.0)
    u = jnp.dot(t, w2_ref[...], preferred_element_type=jnp.float32) + b2_ref[...]
    u = jnp.maximum(_bn(u, ga_ref[...], ba_ref[...]), 0.0)
    u = u * sn_ref[...]
    u = jnp.maximum(_bn(u, gn_ref[...], bn_ref[...]), 0.0)
    return u


def _mid_body(epsp1_ref, h_ref, agg_ref, sn_ref, w1_ref, b1_ref, g1_ref,
              be1_ref, w2_ref, b2_ref, ga_ref, ba_ref, gn_ref, bn_ref,
              hin_ref, o_ref):
    u = _layer_core(epsp1_ref, h_ref, agg_ref, sn_ref, w1_ref, b1_ref, g1_ref,
                    be1_ref, w2_ref, b2_ref, ga_ref, ba_ref, gn_ref, bn_ref)
    o_ref[...] = u + hin_ref[...]


def _last_body(epsp1_ref, h_ref, agg_ref, sn_ref, w1_ref, b1_ref, g1_ref,
               be1_ref, w2_ref, b2_ref, ga_ref, ba_ref, gn_ref, bn_ref,
               hin_ref, wro_ref, wpred_ref, bp_ref, o_ref):
    u = _layer_core(epsp1_ref, h_ref, agg_ref, sn_ref, w1_ref, b1_ref, g1_ref,
                    be1_ref, w2_ref, b2_ref, ga_ref, ba_ref, gn_ref, bn_ref)
    hf = u + hin_ref[...]
    hm = jnp.mean(hf, axis=0, keepdims=True)
    t1 = jnp.dot(hm, wro_ref[...], preferred_element_type=jnp.float32)
    o_ref[...] = jnp.dot(t1, wpred_ref[...],
                         preferred_element_type=jnp.float32) + bp_ref[...]


def kernel(g, h, e, snorm_n, snorm_e, W_embed, eps, W1, b1, gamma1, beta1,
           W2, b2, gamma_a, beta_a, gamma_n, beta_n, W_ro, W_pred, b_pred):
    info = plsc.get_sparse_core_info()
    nc, ns = info.num_cores, info.num_subcores
    total_chunks = -(-E // (ns * CH))
    nch0 = max(NB, round(total_chunks * 0.655))
    nch1 = total_chunks + 1 - nch0
    nchmax = max(nch0, nch1)
    agg_fn = _make_agg(nc, ns, nch0, nch1)

    src = g[0].astype(jnp.int32)
    dst = g[1].astype(jnp.int32)
    e0 = ns * CH * nch0
    e1 = ns * CH * nch1
    pad = e0 + e1 - E
    src_p = jnp.concatenate([src, jnp.zeros((pad,), jnp.int32)])
    dst_p = jnp.concatenate([dst, jnp.full((pad,), N, jnp.int32)])

    def to4(x):
        a = x[:e0].reshape(ns, nch0, CH)
        b = x[e0:].reshape(ns, nch1, CH)
        a = jnp.pad(a, ((0, 0), (0, nchmax - nch0), (0, 0)))
        b = jnp.pad(b, ((0, 0), (0, nchmax - nch1), (0, 0)))
        return jnp.stack([a, b])

    src3 = to4(src_p)
    dst3 = to4(dst_p)

    hcur = pl.pallas_call(
        _embed_body,
        out_shape=jax.ShapeDtypeStruct((N, D), jnp.float32),
    )(h.astype(jnp.float32), W_embed)
    h0 = hcur

    score = None
    for i in range(L):
        agg2 = agg_fn(src3, dst3, hcur)
        epsp1 = (1.0 + eps[i]).reshape(1, 1)
        args = (epsp1, hcur, agg2, snorm_n, W1[i], b1[i].reshape(1, D),
                gamma1[i].reshape(1, D), beta1[i].reshape(1, D), W2[i],
                b2[i].reshape(1, D), gamma_a[i].reshape(1, D),
                beta_a[i].reshape(1, D), gamma_n[i].reshape(1, D),
                beta_n[i].reshape(1, D))
        if i < L - 1:
            hcur = pl.pallas_call(
                _mid_body,
                out_shape=jax.ShapeDtypeStruct((N, D), jnp.float32),
            )(*args, h0)
        else:
            score = pl.pallas_call(
                _last_body,
                out_shape=jax.ShapeDtypeStruct((1, C), jnp.float32),
            )(*args, h0, W_ro, W_pred, b_pred.reshape(1, C))
    return score
